# baseline (speedup 1.0000x reference)
import functools

import jax
import jax.numpy as jnp
from jax import lax
from jax.experimental import pallas as pl
from jax.experimental.pallas import tpu as pltpu
from jax.experimental.pallas import tpu_sc as plsc

N_USERS = 5000
N_ITEMS = 5000
N_NODES = N_USERS + N_ITEMS
EMB = 64
REV = 64
NW = 32
CHUNK = 80
GB = 128

_sc_mesh = functools.partial(
    plsc.VectorSubcoreMesh, core_axis_name="c", subcore_axis_name="s")


def _build_u_body(w_ref, wc_ref, wd_ref, ci_ref, u0, u1, u2, u3, u4, u5, u6,
                  u7):
    ci = ci_ref[...]
    blk = ci.shape[0]
    pieces = [w_ref[r] * ci for r in range(5)]
    pieces += [wc_ref[r] * ci for r in range(5)]
    pieces += [wd_ref[r] * ci for r in range(5)]
    u = jnp.concatenate(pieces, axis=1)
    outs = [u0, u1, u2, u3, u4, u5, u6]
    for b in range(7):
        outs[b][...] = u[:, 128 * b:128 * (b + 1)]
    u7[...] = jnp.concatenate(
        [jnp.broadcast_to(ci, (blk, 64)), u[:, 896:960]], axis=1)


def _build_u(weight, weight_com, weight_dis, ci):
    n = ci.shape[0]
    blk = 2000
    w_spec = pl.BlockSpec((5, blk, EMB), lambda i: (0, i, 0))
    out_shapes = [jax.ShapeDtypeStruct((n, 128), jnp.float32)] * 8
    out_specs = [pl.BlockSpec((blk, 128), lambda i: (i, 0))] * 8
    return pl.pallas_call(
        _build_u_body,
        grid=(n // blk,),
        in_specs=[w_spec, w_spec, w_spec,
                  pl.BlockSpec((blk, 1), lambda i: (i, 0))],
        out_specs=out_specs,
        out_shape=out_shapes,
    )(weight, weight_com, weight_dis, ci)


def _build_rfg_sc(ci128, review_feat, src):
    e = src.shape[0]
    e_per_w = e // NW
    C = 40
    n_ch = e_per_w // C
    NB = 4

    @functools.partial(
        pl.kernel,
        mesh=_sc_mesh(),
        out_type=jax.ShapeDtypeStruct((e, 128), jnp.float32),
        scratch_types=(
            [pltpu.VMEM((C,), jnp.int32)] * NB
            + [pltpu.VMEM((C, 128), jnp.float32)] * NB
            + [pltpu.VMEM((C, REV), jnp.float32)] * NB
            + [pltpu.VMEM((C, 128), jnp.float32)] * NB
            + [pltpu.SemaphoreType.DMA] * (4 * NB)),
    )
    def k(ci_hbm, rf_hbm, src_hbm, rfg_hbm, *scr):
        idx = scr[0:NB]
        civ = scr[NB:2 * NB]
        rfv = scr[2 * NB:3 * NB]
        outv = scr[3 * NB:4 * NB]
        semi = scr[4 * NB:5 * NB]
        semg = scr[5 * NB:6 * NB]
        semr = scr[6 * NB:7 * NB]
        semw = scr[7 * NB:8 * NB]
        wid = lax.axis_index("s") * 2 + lax.axis_index("c")
        base = wid * e_per_w

        def prime(c, b):
            off = jnp.minimum(base + C * c, e - C)
            pltpu.async_copy(src_hbm.at[pl.ds(off, C)], idx[b], semi[b])

        def start_fetch(c, b):
            off = base + C * c
            pltpu.make_async_copy(src_hbm.at[pl.ds(0, C)], idx[b],
                                  semi[b]).wait()
            pltpu.async_copy(ci_hbm.at[idx[b]], civ[b], semg[b])
            pltpu.async_copy(rf_hbm.at[pl.ds(off, C)], rfv[b], semr[b])

        def compute_write(c, b):
            off = base + C * c
            pltpu.make_async_copy(ci_hbm.at[idx[b]], civ[b], semg[b]).wait()
            pltpu.make_async_copy(rf_hbm.at[pl.ds(0, C)], rfv[b],
                                  semr[b]).wait()
            for j in range(C):
                for q in range(4):
                    sl = pl.ds(16 * q, 16)
                    sh = pl.ds(64 + 16 * q, 16)
                    outv[b][j, sl] = rfv[b][j, sl] * civ[b][j, sl]
                    outv[b][j, sh] = civ[b][j, sh]
            pltpu.async_copy(outv[b], rfg_hbm.at[pl.ds(off, C)], semw[b])

        def reuse(c, b):
            pltpu.make_async_copy(outv[b], rfg_hbm.at[pl.ds(0, C)],
                                  semw[b]).wait()
            prime(c, b)

        for b in range(NB):
            prime(b, b)

        def body(i, _):
            ca = NB * i
            for b in range(NB):
                start_fetch(ca + b, b)
            for b in range(NB):
                compute_write(ca + b, b)
            for b in range(NB):
                reuse(ca + NB + b, b)
            return 0

        lax.fori_loop(0, n_ch // NB, body, 0)
        tail = n_ch % NB
        for b in range(tail):
            start_fetch(n_ch - tail + b, b)
        for b in range(tail):
            compute_write(n_ch - tail + b, b)
        for b in range(tail):
            pltpu.make_async_copy(outv[b], rfg_hbm.at[pl.ds(0, C)],
                                  semw[b]).wait()
        for b in range(tail, NB):
            pltpu.make_async_copy(src_hbm.at[pl.ds(0, C)], idx[b],
                                  semi[b]).wait()

    return k(ci128, review_feat, src)


def _segment_sums(src, dst, us, rfg, zeros_a):
    e = src.shape[0]
    e_per_t = e // 16
    n_ch = e_per_t // CHUNK
    n_pad = 10240
    rows_per_t = n_pad // 16
    out_type = [jax.ShapeDtypeStruct((n_pad, 128), jnp.float32)] * 8

    NB = 4

    @functools.partial(
        pl.kernel,
        mesh=_sc_mesh(),
        out_type=out_type,
        scratch_types=(
            [pltpu.VMEM((CHUNK,), jnp.int32)] * (2 * NB)
            + [pltpu.VMEM((CHUNK, 128), jnp.float32)] * NB
            + [pltpu.VMEM_SHARED((n_pad, 128), jnp.float32)]
            + [pltpu.SemaphoreType.DMA] * (3 * NB)),
    )
    def k(src_hbm, dst_hbm, u0, u1, u2, u3, u4, u5, u6, rfg_hbm,
          za_hbm,
          z0, z1, z2, z3, z4, z5, z6, s_out,
          *scr):
        idxs = scr[0:NB]
        idxd = scr[NB:2 * NB]
        stage = scr[2 * NB:3 * NB]
        acc = scr[3 * NB]
        semi = scr[3 * NB + 1:4 * NB + 1]
        semg = scr[4 * NB + 1:5 * NB + 1]
        sems = scr[5 * NB + 1:6 * NB + 1]
        core = lax.axis_index("c")
        tid = lax.axis_index("s")
        row0 = tid * rows_per_t
        ebase = tid * e_per_t

        def run_pass(tbl_hbm, out_hbm, is_gather, c0, c1):
            nch = c1 - c0
            pltpu.sync_copy(za_hbm, acc.at[pl.ds(row0, rows_per_t)])
            plsc.subcore_barrier()

            def prime(c, b):
                off = jnp.minimum(ebase + CHUNK * c, e - CHUNK)
                pltpu.async_copy(src_hbm.at[pl.ds(off, CHUNK)],
                                 idxs[b], semi[b])
                pltpu.async_copy(dst_hbm.at[pl.ds(off, CHUNK)],
                                 idxd[b], semi[b])

            def wait_idx(b):
                pltpu.make_async_copy(src_hbm.at[pl.ds(0, CHUNK)],
                                      idxs[b], semi[b]).wait()
                pltpu.make_async_copy(dst_hbm.at[pl.ds(0, CHUNK)],
                                      idxd[b], semi[b]).wait()

            def start_fetch(c, b):
                if is_gather:
                    pltpu.async_copy(tbl_hbm.at[idxs[b]], stage[b], semg[b])
                else:
                    off = ebase + CHUNK * c
                    pltpu.async_copy(tbl_hbm.at[pl.ds(off, CHUNK)],
                                     stage[b], semg[b])

            def wait_fetch(c, b):
                pltpu.make_async_copy(rfg_hbm.at[pl.ds(0, CHUNK)],
                                      stage[b], semg[b]).wait()

            def start_scatter(b):
                pltpu.async_copy(stage[b], acc.at[idxd[b]], sems[b],
                                 add=True)

            def wait_scatter(b):
                pltpu.make_async_copy(stage[b], acc.at[idxd[b]],
                                      sems[b]).wait()

            for b in range(NB):
                prime(c0 + b, b)

            def body(i, _):
                ca = c0 + NB * i
                for b in range(NB):
                    wait_idx(b)
                    start_fetch(ca + b, b)
                for b in range(NB):
                    wait_fetch(ca + b, b)
                    start_scatter(b)
                for b in range(NB):
                    wait_scatter(b)
                    prime(ca + NB + b, b)
                return 0

            lax.fori_loop(0, nch // NB, body, 0)
            tail = nch % NB
            for b in range(tail):
                wait_idx(b)
                start_fetch(c1 - tail + b, b)
            for b in range(tail):
                wait_fetch(c1 - tail + b, b)
                start_scatter(b)
            for b in range(tail):
                wait_scatter(b)
            for b in range(tail, NB):
                wait_idx(b)
            plsc.subcore_barrier()
            pltpu.sync_copy(acc.at[pl.ds(row0, rows_per_t)],
                            out_hbm.at[pl.ds(row0, rows_per_t)])
            plsc.subcore_barrier()

        @pl.when(core == 0)
        def _():
            run_pass(u0, z0, True, 0, n_ch)
            run_pass(u1, z1, True, 0, n_ch)
            run_pass(u2, z2, True, 0, n_ch)
            run_pass(u3, z3, True, 0, n_ch)

        @pl.when(core == 1)
        def _():
            run_pass(u4, z4, True, 0, n_ch)
            run_pass(u5, z5, True, 0, n_ch)
            run_pass(u6, z6, True, 0, n_ch)
            run_pass(rfg_hbm, s_out, False, 0, n_ch)

    return k(src, dst, *us, rfg, zeros_a)


def _row_gathers(tables, users, iid):
    b = users.shape[0]
    nt = len(tables)
    out_type = [jax.ShapeDtypeStruct((b, 128), jnp.float32)] * (2 * nt)

    @functools.partial(
        pl.kernel,
        mesh=_sc_mesh(),
        out_type=out_type,
        scratch_types=[pltpu.VMEM((GB,), jnp.int32),
                       pltpu.VMEM((GB,), jnp.int32),
                       pltpu.VMEM((GB, 128), jnp.float32),
                       pltpu.SemaphoreType.DMA],
    )
    def k(*refs):
        tbls = refs[:nt]
        users_hbm, iid_hbm = refs[nt], refs[nt + 1]
        outs = refs[nt + 2:nt + 2 + 2 * nt]
        idx_u = refs[nt + 2 + 2 * nt]
        idx_i = refs[nt + 3 + 2 * nt]
        stg = refs[nt + 4 + 2 * nt]
        sem = refs[-1]
        wid = lax.axis_index("s") * 2 + lax.axis_index("c")
        off = wid * GB
        pltpu.sync_copy(users_hbm.at[pl.ds(off, GB)], idx_u)
        pltpu.sync_copy(iid_hbm.at[pl.ds(off, GB)], idx_i)
        for j in range(nt):
            pltpu.async_copy(tbls[j].at[idx_u], stg, sem).wait()
            pltpu.sync_copy(stg, outs[2 * j].at[pl.ds(off, GB)])
            pltpu.async_copy(tbls[j].at[idx_i], stg, sem).wait()
            pltpu.sync_copy(stg, outs[2 * j + 1].at[pl.ds(off, GB)])

    return k(*tables, users, iid)


def _dense_body(*refs):
    (zu0, zu1, zu2, zu3, zu4, zu5, zu6, su, cu,
     zi0, zi1, zi2, zi3, zi4, zi5, zi6, si, cii,
     wint_ref, wrev_ref,
     fuw, fub, fiw, fib, fucw, fucb, ficw, ficb,
     fudw, fudb, fidw, fidb, furw, furb, firw, firb,
     piw1, piw2, prw1, prw2, pcw1, pcw2, pdw1, pdw2,
     pjw1, pjw2, out_ref) = refs
    f32 = jnp.float32

    def mm(a, b):
        return jnp.dot(a, b, preferred_element_type=f32)

    def side(zrefs, s_ref, c_ref, fw, fb, fcw, fcb, fdw, fdb, frw, frb):
        z = jnp.concatenate([r[...] for r in zrefs] + [s_ref[:, 64:128]],
                            axis=1)
        s = s_ref[:, :64]
        c = c_ref[:, :1]
        pieces = []
        rev_pieces = []
        for r in range(5):
            pieces.append(z[:, 64 * r:64 * (r + 1)])
            pieces.append(mm(s, wint_ref[r]))
            rev_pieces.append(mm(s, wrev_ref[r]))
        fall = jnp.concatenate(pieces, axis=1) * c
        fid = mm(fall, fw[...]) + fb[...]
        fcom = mm(z[:, 320:640] * c, fcw[...]) + fcb[...]
        fdis = mm(z[:, 640:960] * c, fdw[...]) + fdb[...]
        frev = mm(jnp.concatenate(rev_pieces, axis=1) * c, frw[...]) + frb[...]
        return fid, fcom, fdis, frev

    fid_u, fc_u, fd_u, fr_u = side(
        (zu0, zu1, zu2, zu3, zu4, zu5, zu6), su, cu,
        fuw, fub, fucw, fucb, fudw, fudb, furw, furb)
    fid_i, fc_i, fd_i, fr_i = side(
        (zi0, zi1, zi2, zi3, zi4, zi5, zi6), si, cii,
        fiw, fib, ficw, ficb, fidw, fidb, firw, firb)

    def head(a, b, w1, w2):
        z = a * b
        return mm(jax.nn.relu(mm(z, w1[...])), w2[...])

    oi = head(fid_u, fid_i, piw1, piw2)
    orv = head(fr_u, fr_i, prw1, prw2)
    oc = head(fc_u, fc_i, pcw1, pcw2)
    od = head(fd_u, fd_i, pdw1, pdw2)
    sim = jnp.sum(mm(fc_u, pjw1[...]) * mm(fr_u, pjw2[...]), axis=1,
                  keepdims=True)
    out_ref[...] = jnp.concatenate([oi, orv, oc, od, sim], axis=1)


def _dense_block(u_parts, i_parts, wint, wrev, fcs, preds, proj):
    b = u_parts[0].shape[0]
    blk = 512
    args = list(u_parts) + list(i_parts) + [wint, wrev] + fcs + preds + proj

    def whole(a):
        return pl.BlockSpec(a.shape, lambda i: (0,) * a.ndim)

    def rows(a):
        return pl.BlockSpec((blk, a.shape[1]), lambda i: (i, 0))

    in_specs = ([rows(a) for a in u_parts] + [rows(a) for a in i_parts]
                + [whole(a) for a in args[18:]])
    return pl.pallas_call(
        _dense_body,
        grid=(b // blk,),
        in_specs=in_specs,
        out_specs=pl.BlockSpec((blk, 21), lambda i: (i, 0)),
        out_shape=jax.ShapeDtypeStruct((b, 21), jnp.float32),
    )(*args)


def kernel(edge_index, users, items, ci, review_feat, weight, weight_com,
           weight_dis, review_w_int, review_w_rev,
           fc_user_w, fc_user_b, fc_item_w, fc_item_b,
           fc_user_com_w, fc_user_com_b, fc_item_com_w, fc_item_com_b,
           fc_user_dis_w, fc_user_dis_b, fc_item_dis_w, fc_item_dis_b,
           fc_user_rev_w, fc_user_rev_b, fc_item_rev_w, fc_item_rev_b,
           pred_int_w1, pred_int_w2, pred_rev_w1, pred_rev_w2,
           pred_com_w1, pred_com_w2, pred_dis_w1, pred_dis_w2,
           proj_w1, proj_w2):
    src = edge_index[0]
    dst = edge_index[1]
    iid = items + N_USERS

    us = _build_u(weight, weight_com, weight_dis, ci)
    pack = us[7]
    rfg = _build_rfg_sc(pack, review_feat, src)

    zeros_a = jnp.zeros((640, 128), jnp.float32)
    zs = _segment_sums(src, dst, us[:7], rfg, zeros_a)

    gathered = _row_gathers(list(zs) + [pack], users, iid)
    u_parts = [gathered[2 * j] for j in range(9)]
    i_parts = [gathered[2 * j + 1] for j in range(9)]

    fcs = [fc_user_w, fc_user_b.reshape(1, -1), fc_item_w,
           fc_item_b.reshape(1, -1),
           fc_user_com_w, fc_user_com_b.reshape(1, -1), fc_item_com_w,
           fc_item_com_b.reshape(1, -1),
           fc_user_dis_w, fc_user_dis_b.reshape(1, -1), fc_item_dis_w,
           fc_item_dis_b.reshape(1, -1),
           fc_user_rev_w, fc_user_rev_b.reshape(1, -1), fc_item_rev_w,
           fc_item_rev_b.reshape(1, -1)]
    preds = [pred_int_w1, pred_int_w2, pred_rev_w1, pred_rev_w2,
             pred_com_w1, pred_com_w2, pred_dis_w1, pred_dis_w2]
    proj = [proj_w1, proj_w2]
    return _dense_block(u_parts, i_parts, review_w_int, review_w_rev, fcs,
                        preds, proj)


# speedup vs baseline: 1.0034x; 1.0034x over previous
import functools

import jax
import jax.numpy as jnp
from jax import lax
from jax.experimental import pallas as pl
from jax.experimental.pallas import tpu as pltpu
from jax.experimental.pallas import tpu_sc as plsc

N_USERS = 5000
N_ITEMS = 5000
N_NODES = N_USERS + N_ITEMS
EMB = 64
REV = 64
NW = 32
CHUNK = 80
GB = 128

_sc_mesh = functools.partial(
    plsc.VectorSubcoreMesh, core_axis_name="c", subcore_axis_name="s")


def _build_u_body(w_ref, wc_ref, wd_ref, ci_ref, u0, u1, u2, u3, u4, u5, u6,
                  u7):
    ci = ci_ref[...]
    blk = ci.shape[0]
    pieces = [w_ref[r] * ci for r in range(5)]
    pieces += [wc_ref[r] * ci for r in range(5)]
    pieces += [wd_ref[r] * ci for r in range(5)]
    u = jnp.concatenate(pieces, axis=1)
    outs = [u0, u1, u2, u3, u4, u5, u6]
    for b in range(7):
        outs[b][...] = u[:, 128 * b:128 * (b + 1)]
    u7[...] = jnp.concatenate(
        [u[:, 896:960], jnp.zeros((blk, 64), jnp.float32)], axis=1)


def _build_u(weight, weight_com, weight_dis, ci):
    n = ci.shape[0]
    blk = 2000
    w_spec = pl.BlockSpec((5, blk, EMB), lambda i: (0, i, 0))
    out_shapes = [jax.ShapeDtypeStruct((n, 128), jnp.float32)] * 8
    out_specs = [pl.BlockSpec((blk, 128), lambda i: (i, 0))] * 8
    return pl.pallas_call(
        _build_u_body,
        grid=(n // blk,),
        in_specs=[w_spec, w_spec, w_spec,
                  pl.BlockSpec((blk, 1), lambda i: (i, 0))],
        out_specs=out_specs,
        out_shape=out_shapes,
    )(weight, weight_com, weight_dis, ci)


def _build_ci128_body(ci_ref, out_ref):
    out_ref[...] = jnp.broadcast_to(ci_ref[...], out_ref.shape)


def _build_ci128(ci):
    n = ci.shape[0]
    blk = 2000
    return pl.pallas_call(
        _build_ci128_body,
        grid=(n // blk,),
        in_specs=[pl.BlockSpec((blk, 1), lambda i: (i, 0))],
        out_specs=pl.BlockSpec((blk, 128), lambda i: (i, 0)),
        out_shape=jax.ShapeDtypeStruct((n, 128), jnp.float32),
    )(ci)


def _build_rfg_sc(ci128, review_feat, src):
    e = src.shape[0]
    e_per_w = e // NW
    C = 40
    n_ch = e_per_w // C
    NB = 4

    @functools.partial(
        pl.kernel,
        mesh=_sc_mesh(),
        out_type=jax.ShapeDtypeStruct((e, 128), jnp.float32),
        scratch_types=(
            [pltpu.VMEM((C,), jnp.int32)] * NB
            + [pltpu.VMEM((C, 128), jnp.float32)] * NB
            + [pltpu.VMEM((C, REV), jnp.float32)] * NB
            + [pltpu.VMEM((C, 128), jnp.float32)] * NB
            + [pltpu.SemaphoreType.DMA] * (4 * NB)),
    )
    def k(ci_hbm, rf_hbm, src_hbm, rfg_hbm, *scr):
        idx = scr[0:NB]
        civ = scr[NB:2 * NB]
        rfv = scr[2 * NB:3 * NB]
        outv = scr[3 * NB:4 * NB]
        semi = scr[4 * NB:5 * NB]
        semg = scr[5 * NB:6 * NB]
        semr = scr[6 * NB:7 * NB]
        semw = scr[7 * NB:8 * NB]
        wid = lax.axis_index("s") * 2 + lax.axis_index("c")
        base = wid * e_per_w

        zero16 = jnp.zeros((16,), jnp.float32)
        for b in range(NB):
            for j in range(C):
                for q in range(4):
                    outv[b][j, pl.ds(64 + 16 * q, 16)] = zero16

        def prime(c, b):
            off = jnp.minimum(base + C * c, e - C)
            pltpu.async_copy(src_hbm.at[pl.ds(off, C)], idx[b], semi[b])

        def start_fetch(c, b):
            off = base + C * c
            pltpu.make_async_copy(src_hbm.at[pl.ds(0, C)], idx[b],
                                  semi[b]).wait()
            pltpu.async_copy(ci_hbm.at[idx[b]], civ[b], semg[b])
            pltpu.async_copy(rf_hbm.at[pl.ds(off, C)], rfv[b], semr[b])

        def compute_write(c, b):
            off = base + C * c
            pltpu.make_async_copy(ci_hbm.at[idx[b]], civ[b], semg[b]).wait()
            pltpu.make_async_copy(rf_hbm.at[pl.ds(0, C)], rfv[b],
                                  semr[b]).wait()
            for j in range(C):
                for q in range(4):
                    sl = pl.ds(16 * q, 16)
                    outv[b][j, sl] = rfv[b][j, sl] * civ[b][j, sl]
            pltpu.async_copy(outv[b], rfg_hbm.at[pl.ds(off, C)], semw[b])

        def reuse(c, b):
            pltpu.make_async_copy(outv[b], rfg_hbm.at[pl.ds(0, C)],
                                  semw[b]).wait()
            prime(c, b)

        for b in range(NB):
            prime(b, b)

        def body(i, _):
            ca = NB * i
            for b in range(NB):
                start_fetch(ca + b, b)
            for b in range(NB):
                compute_write(ca + b, b)
            for b in range(NB):
                reuse(ca + NB + b, b)
            return 0

        lax.fori_loop(0, n_ch // NB, body, 0)
        tail = n_ch % NB
        for b in range(tail):
            start_fetch(n_ch - tail + b, b)
        for b in range(tail):
            compute_write(n_ch - tail + b, b)
        for b in range(tail):
            pltpu.make_async_copy(outv[b], rfg_hbm.at[pl.ds(0, C)],
                                  semw[b]).wait()
        for b in range(tail, NB):
            pltpu.make_async_copy(src_hbm.at[pl.ds(0, C)], idx[b],
                                  semi[b]).wait()

    return k(ci128, review_feat, src)


def _segment_sums(src, dst, us, rfg, zeros_a):
    e = src.shape[0]
    e_per_t = e // 16
    n_ch = e_per_t // CHUNK
    n_pad = 10240
    rows_per_t = n_pad // 16
    rf_split = (n_ch // 2) + 1

    out_type = [jax.ShapeDtypeStruct((n_pad, 128), jnp.float32)] * 10

    NB = 4

    @functools.partial(
        pl.kernel,
        mesh=_sc_mesh(),
        out_type=out_type,
        scratch_types=(
            [pltpu.VMEM((CHUNK,), jnp.int32)] * (2 * NB)
            + [pltpu.VMEM((CHUNK, 128), jnp.float32)] * NB
            + [pltpu.VMEM_SHARED((n_pad, 128), jnp.float32)]
            + [pltpu.SemaphoreType.DMA] * (3 * NB)),
    )
    def k(src_hbm, dst_hbm, u0, u1, u2, u3, u4, u5, u6, u7, rfg_hbm,
          za_hbm,
          z0, z1, z2, z3, z4, z5, z6, z7, s_a, s_b,
          *scr):
        idxs = scr[0:NB]
        idxd = scr[NB:2 * NB]
        stage = scr[2 * NB:3 * NB]
        acc = scr[3 * NB]
        semi = scr[3 * NB + 1:4 * NB + 1]
        semg = scr[4 * NB + 1:5 * NB + 1]
        sems = scr[5 * NB + 1:6 * NB + 1]
        core = lax.axis_index("c")
        tid = lax.axis_index("s")
        row0 = tid * rows_per_t
        ebase = tid * e_per_t

        def run_pass(tbl_hbm, out_hbm, is_gather, c0, c1):
            nch = c1 - c0
            pltpu.sync_copy(za_hbm, acc.at[pl.ds(row0, rows_per_t)])
            plsc.subcore_barrier()

            def prime(c, b):
                off = jnp.minimum(ebase + CHUNK * c, e - CHUNK)
                pltpu.async_copy(src_hbm.at[pl.ds(off, CHUNK)],
                                 idxs[b], semi[b])
                pltpu.async_copy(dst_hbm.at[pl.ds(off, CHUNK)],
                                 idxd[b], semi[b])

            def wait_idx(b):
                pltpu.make_async_copy(src_hbm.at[pl.ds(0, CHUNK)],
                                      idxs[b], semi[b]).wait()
                pltpu.make_async_copy(dst_hbm.at[pl.ds(0, CHUNK)],
                                      idxd[b], semi[b]).wait()

            def start_fetch(c, b):
                if is_gather:
                    pltpu.async_copy(tbl_hbm.at[idxs[b]], stage[b], semg[b])
                else:
                    off = ebase + CHUNK * c
                    pltpu.async_copy(tbl_hbm.at[pl.ds(off, CHUNK)],
                                     stage[b], semg[b])

            def wait_fetch(c, b):
                pltpu.make_async_copy(rfg_hbm.at[pl.ds(0, CHUNK)],
                                      stage[b], semg[b]).wait()

            def start_scatter(b):
                pltpu.async_copy(stage[b], acc.at[idxd[b]], sems[b],
                                 add=True)

            def wait_scatter(b):
                pltpu.make_async_copy(stage[b], acc.at[idxd[b]],
                                      sems[b]).wait()

            for b in range(NB):
                prime(c0 + b, b)

            def body(i, _):
                ca = c0 + NB * i
                for b in range(NB):
                    wait_idx(b)
                    start_fetch(ca + b, b)
                for b in range(NB):
                    wait_fetch(ca + b, b)
                    start_scatter(b)
                for b in range(NB):
                    wait_scatter(b)
                    prime(ca + NB + b, b)
                return 0

            lax.fori_loop(0, nch // NB, body, 0)
            tail = nch % NB
            for b in range(tail):
                wait_idx(b)
                start_fetch(c1 - tail + b, b)
            for b in range(tail):
                wait_fetch(c1 - tail + b, b)
                start_scatter(b)
            for b in range(tail):
                wait_scatter(b)
            for b in range(tail, NB):
                wait_idx(b)
            plsc.subcore_barrier()
            pltpu.sync_copy(acc.at[pl.ds(row0, rows_per_t)],
                            out_hbm.at[pl.ds(row0, rows_per_t)])
            plsc.subcore_barrier()

        @pl.when(core == 0)
        def _():
            run_pass(u0, z0, True, 0, n_ch)
            run_pass(u1, z1, True, 0, n_ch)
            run_pass(u2, z2, True, 0, n_ch)
            run_pass(u3, z3, True, 0, n_ch)
            run_pass(rfg_hbm, s_a, False, 0, rf_split)

        @pl.when(core == 1)
        def _():
            run_pass(u4, z4, True, 0, n_ch)
            run_pass(u5, z5, True, 0, n_ch)
            run_pass(u6, z6, True, 0, n_ch)
            run_pass(u7, z7, True, 0, n_ch)
            run_pass(rfg_hbm, s_b, False, rf_split, n_ch)

    return k(src, dst, *us, rfg, zeros_a)


def _row_gathers(tables, users, iid):
    b = users.shape[0]
    nt = len(tables)
    out_type = [jax.ShapeDtypeStruct((b, 128), jnp.float32)] * (2 * nt)

    @functools.partial(
        pl.kernel,
        mesh=_sc_mesh(),
        out_type=out_type,
        scratch_types=[pltpu.VMEM((GB,), jnp.int32),
                       pltpu.VMEM((GB,), jnp.int32),
                       pltpu.VMEM((GB, 128), jnp.float32),
                       pltpu.SemaphoreType.DMA],
    )
    def k(*refs):
        tbls = refs[:nt]
        users_hbm, iid_hbm = refs[nt], refs[nt + 1]
        outs = refs[nt + 2:nt + 2 + 2 * nt]
        idx_u = refs[nt + 2 + 2 * nt]
        idx_i = refs[nt + 3 + 2 * nt]
        stg = refs[nt + 4 + 2 * nt]
        sem = refs[-1]
        wid = lax.axis_index("s") * 2 + lax.axis_index("c")
        off = wid * GB
        pltpu.sync_copy(users_hbm.at[pl.ds(off, GB)], idx_u)
        pltpu.sync_copy(iid_hbm.at[pl.ds(off, GB)], idx_i)
        for j in range(nt):
            pltpu.async_copy(tbls[j].at[idx_u], stg, sem).wait()
            pltpu.sync_copy(stg, outs[2 * j].at[pl.ds(off, GB)])
            pltpu.async_copy(tbls[j].at[idx_i], stg, sem).wait()
            pltpu.sync_copy(stg, outs[2 * j + 1].at[pl.ds(off, GB)])

    return k(*tables, users, iid)


def _dense_body(*refs):
    (zu0, zu1, zu2, zu3, zu4, zu5, zu6, zu7, sua, sub, cu,
     zi0, zi1, zi2, zi3, zi4, zi5, zi6, zi7, sia, sib, cii,
     wint_ref, wrev_ref,
     fuw, fub, fiw, fib, fucw, fucb, ficw, ficb,
     fudw, fudb, fidw, fidb, furw, furb, firw, firb,
     piw1, piw2, prw1, prw2, pcw1, pcw2, pdw1, pdw2,
     pjw1, pjw2, out_ref) = refs
    f32 = jnp.float32

    def mm(a, b):
        return jnp.dot(a, b, preferred_element_type=f32)

    def side(zrefs, z7_ref, sa_ref, sb_ref, c_ref, fw, fb, fcw, fcb, fdw,
             fdb, frw, frb):
        z = jnp.concatenate([r[...] for r in zrefs] + [z7_ref[:, :64]],
                            axis=1)
        s = sa_ref[:, :64] + sb_ref[:, :64]
        c = c_ref[:, :1]
        pieces = []
        rev_pieces = []
        for r in range(5):
            pieces.append(z[:, 64 * r:64 * (r + 1)])
            pieces.append(mm(s, wint_ref[r]))
            rev_pieces.append(mm(s, wrev_ref[r]))
        fall = jnp.concatenate(pieces, axis=1) * c
        fid = mm(fall, fw[...]) + fb[...]
        fcom = mm(z[:, 320:640] * c, fcw[...]) + fcb[...]
        fdis = mm(z[:, 640:960] * c, fdw[...]) + fdb[...]
        frev = mm(jnp.concatenate(rev_pieces, axis=1) * c, frw[...]) + frb[...]
        return fid, fcom, fdis, frev

    fid_u, fc_u, fd_u, fr_u = side(
        (zu0, zu1, zu2, zu3, zu4, zu5, zu6), zu7, sua, sub, cu,
        fuw, fub, fucw, fucb, fudw, fudb, furw, furb)
    fid_i, fc_i, fd_i, fr_i = side(
        (zi0, zi1, zi2, zi3, zi4, zi5, zi6), zi7, sia, sib, cii,
        fiw, fib, ficw, ficb, fidw, fidb, firw, firb)

    def head(a, b, w1, w2):
        z = a * b
        return mm(jax.nn.relu(mm(z, w1[...])), w2[...])

    oi = head(fid_u, fid_i, piw1, piw2)
    orv = head(fr_u, fr_i, prw1, prw2)
    oc = head(fc_u, fc_i, pcw1, pcw2)
    od = head(fd_u, fd_i, pdw1, pdw2)
    sim = jnp.sum(mm(fc_u, pjw1[...]) * mm(fr_u, pjw2[...]), axis=1,
                  keepdims=True)
    out_ref[...] = jnp.concatenate([oi, orv, oc, od, sim], axis=1)


def _dense_block(u_parts, i_parts, wint, wrev, fcs, preds, proj):
    b = u_parts[0].shape[0]
    blk = 512
    args = list(u_parts) + list(i_parts) + [wint, wrev] + fcs + preds + proj

    def whole(a):
        return pl.BlockSpec(a.shape, lambda i: (0,) * a.ndim)

    def rows(a):
        return pl.BlockSpec((blk, a.shape[1]), lambda i: (i, 0))

    in_specs = ([rows(a) for a in u_parts] + [rows(a) for a in i_parts]
                + [whole(a) for a in args[22:]])
    return pl.pallas_call(
        _dense_body,
        grid=(b // blk,),
        in_specs=in_specs,
        out_specs=pl.BlockSpec((blk, 21), lambda i: (i, 0)),
        out_shape=jax.ShapeDtypeStruct((b, 21), jnp.float32),
    )(*args)


def kernel(edge_index, users, items, ci, review_feat, weight, weight_com,
           weight_dis, review_w_int, review_w_rev,
           fc_user_w, fc_user_b, fc_item_w, fc_item_b,
           fc_user_com_w, fc_user_com_b, fc_item_com_w, fc_item_com_b,
           fc_user_dis_w, fc_user_dis_b, fc_item_dis_w, fc_item_dis_b,
           fc_user_rev_w, fc_user_rev_b, fc_item_rev_w, fc_item_rev_b,
           pred_int_w1, pred_int_w2, pred_rev_w1, pred_rev_w2,
           pred_com_w1, pred_com_w2, pred_dis_w1, pred_dis_w2,
           proj_w1, proj_w2):
    src = edge_index[0]
    dst = edge_index[1]
    iid = items + N_USERS

    ci128 = _build_ci128(ci)
    rfg = _build_rfg_sc(ci128, review_feat, src)
    us = _build_u(weight, weight_com, weight_dis, ci)

    zeros_a = jnp.zeros((640, 128), jnp.float32)
    zs = _segment_sums(src, dst, us, rfg, zeros_a)

    gathered = _row_gathers(list(zs) + [ci128], users, iid)
    u_parts = [gathered[2 * j] for j in range(11)]
    i_parts = [gathered[2 * j + 1] for j in range(11)]

    fcs = [fc_user_w, fc_user_b.reshape(1, -1), fc_item_w,
           fc_item_b.reshape(1, -1),
           fc_user_com_w, fc_user_com_b.reshape(1, -1), fc_item_com_w,
           fc_item_com_b.reshape(1, -1),
           fc_user_dis_w, fc_user_dis_b.reshape(1, -1), fc_item_dis_w,
           fc_item_dis_b.reshape(1, -1),
           fc_user_rev_w, fc_user_rev_b.reshape(1, -1), fc_item_rev_w,
           fc_item_rev_b.reshape(1, -1)]
    preds = [pred_int_w1, pred_int_w2, pred_rev_w1, pred_rev_w2,
             pred_com_w1, pred_com_w2, pred_dis_w1, pred_dis_w2]
    proj = [proj_w1, proj_w2]
    return _dense_block(u_parts, i_parts, review_w_int, review_w_rev, fcs,
                        preds, proj)
